# baseline (speedup 1.0000x reference)
import functools

import jax
import jax.numpy as jnp
from jax import lax
from jax.experimental import pallas as pl
from jax.experimental.pallas import tpu as pltpu
from jax.experimental.pallas import tpu_sc as plsc

_B, _N, _F = 4, 1024, 256
_H, _M, _D = 8, 2048, 32
_NC, _NS, _L = 2, 16, 16


def _main_body(x_ref, c_ref, out_ref, idx_ref, loss_ref):
    b = pl.program_id(0)
    iota = jax.lax.broadcasted_iota(jnp.int32, (_N, _M), 1)

    @pl.when(b == 0)
    def _():
        loss_ref[...] = jnp.zeros_like(loss_ref)

    csums = []
    for h in range(_H):
        c = c_ref[h]
        l2_c = jnp.sum(c * c, axis=1)
        q = x_ref[0, :, h * _D:(h + 1) * _D]
        l2_q = jnp.sum(q * q, axis=1, keepdims=True)
        dot = jax.lax.dot_general(q, c, (((1,), (1,)), ((), ())),
                                  preferred_element_type=jnp.float32)
        sim = -(l2_q + l2_c[None, :] - 2.0 * dot)
        row_max = jnp.max(sim, axis=-1, keepdims=True)
        picked = jnp.where(sim == row_max, iota, _M)
        idx_ref[0, h, :] = jnp.min(picked, axis=-1).astype(jnp.int32)
        csums.append(jnp.sum(c, axis=0))

    csum_flat = jnp.concatenate(csums)
    out_ref[0] = jnp.broadcast_to(csum_flat[None, :], (_N, _F))
    diff = x_ref[0] - csum_flat[None, :]
    loss_ref[...] += jnp.sum(diff * diff) * (1.0 / (_B * _N * _F))


def _hist_body(idx_hbm, counts_hbm, idx_v, zeros_v, ones_v, shared):
    cid = lax.axis_index("c")
    sid = lax.axis_index("s")
    slab = cid * _NS + sid
    h = slab % _H
    pltpu.sync_copy(idx_hbm.at[pl.ds(slab * _N, _N)], idx_v)

    def _zero(i, carry):
        zeros_v[pl.ds(i * _L, _L)] = jnp.zeros((_L,), jnp.float32)
        return carry

    lax.fori_loop(0, _N // _L, _zero, 0)
    ones_v[...] = jnp.full((_L,), 1.0, jnp.float32)
    pltpu.sync_copy(zeros_v, shared.at[pl.ds(sid * _N, _N)])
    plsc.subcore_barrier()

    base = jnp.full((_L,), h * _M, jnp.int32)

    def _accum(j, carry):
        iv = idx_v[pl.ds(j * _L, _L)] + base
        pltpu.sync_copy(ones_v, shared.at[iv], add=True)
        return carry

    lax.fori_loop(0, _N // _L, _accum, 0)
    plsc.subcore_barrier()

    @pl.when(sid < _H)
    def _():
        pltpu.sync_copy(shared.at[pl.ds(sid * _M, _M)],
                        counts_hbm.at[cid, sid])


def _finish_body(counts_ref, ema_ref, perp_ref, repl_ref):
    counts = counts_ref[0] + counts_ref[1]
    mean = counts * (1.0 / (_B * _N))
    ent = -jnp.sum(mean * jnp.log(mean + 1e-10), axis=1, keepdims=True)
    perp_ref[...] = jnp.broadcast_to(jnp.exp(ent), (_H, 128))
    expired = (ema_ref[...] < 2.0).astype(jnp.int32)
    repl_ref[...] = jnp.broadcast_to(
        jnp.sum(expired, axis=1, keepdims=True), (_H, 128))


def _run_main(x, codebooks, interpret=False):
    return pl.pallas_call(
        _main_body,
        grid=(_B,),
        in_specs=[
            pl.BlockSpec((1, _N, _F), lambda b: (b, 0, 0)),
            pl.BlockSpec((_H, _M, _D), lambda b: (0, 0, 0)),
        ],
        out_specs=[
            pl.BlockSpec((1, _N, _F), lambda b: (b, 0, 0)),
            pl.BlockSpec((1, _H, _N), lambda b: (b, 0, 0)),
            pl.BlockSpec((1, 128), lambda b: (0, 0)),
        ],
        out_shape=[
            jax.ShapeDtypeStruct((_B, _N, _F), jnp.float32),
            jax.ShapeDtypeStruct((_B, _H, _N), jnp.int32),
            jax.ShapeDtypeStruct((1, 128), jnp.float32),
        ],
        interpret=interpret,
    )(x, codebooks)


def _run_hist(idx_flat):
    mesh = plsc.VectorSubcoreMesh(core_axis_name="c", subcore_axis_name="s")
    return pl.kernel(
        _hist_body,
        out_type=jax.ShapeDtypeStruct((_NC, _H, _M), jnp.float32),
        mesh=mesh,
        scratch_types=[
            pltpu.VMEM((_N,), jnp.int32),
            pltpu.VMEM((_N,), jnp.float32),
            pltpu.VMEM((_L,), jnp.float32),
            pltpu.VMEM_SHARED((_H * _M,), jnp.float32),
        ],
    )(idx_flat)


def _run_finish(counts2, ema, interpret=False):
    return pl.pallas_call(
        _finish_body,
        out_shape=[
            jax.ShapeDtypeStruct((_H, 128), jnp.float32),
            jax.ShapeDtypeStruct((_H, 128), jnp.int32),
        ],
        interpret=interpret,
    )(counts2, ema)


@jax.jit
def kernel(x, codebooks, ema_cluster_size):
    out, idx, loss = _run_main(x, codebooks)
    counts2 = _run_hist(idx.reshape(_B * _H * _N))
    perp, repl = _run_finish(counts2, ema_cluster_size)
    return out, idx, loss[0, 0], perp[:, 0], repl[:, 0]


# speedup vs baseline: 1.0520x; 1.0520x over previous
import functools

import jax
import jax.numpy as jnp
from jax import lax
from jax.experimental import pallas as pl
from jax.experimental.pallas import tpu as pltpu
from jax.experimental.pallas import tpu_sc as plsc

_B, _N, _F = 4, 1024, 256
_H, _M, _D = 8, 2048, 32
_NC, _NS, _L = 2, 16, 16


def _main_body(x_ref, c_ref, idx_ref):
    iota = jax.lax.broadcasted_iota(jnp.int32, (_N, _M), 1)
    for h in range(_H):
        c = c_ref[h]
        l2_c = jnp.sum(c * c, axis=1)
        q = x_ref[0, :, h * _D:(h + 1) * _D]
        l2_q = jnp.sum(q * q, axis=1, keepdims=True)
        dot = jax.lax.dot_general(q, c, (((1,), (1,)), ((), ())),
                                  preferred_element_type=jnp.float32)
        sim = 2.0 * dot - (l2_q + l2_c[None, :])
        row_max = jnp.max(sim, axis=-1, keepdims=True)
        picked = jnp.where(sim == row_max, iota, _M)
        idx_ref[0, h, :] = jnp.min(picked, axis=-1).astype(jnp.int32)


def _out_loss_body(x_ref, c_ref, out_ref, loss_ref):
    b = pl.program_id(0)

    @pl.when(b == 0)
    def _():
        loss_ref[...] = jnp.zeros_like(loss_ref)

    csums = [jnp.sum(c_ref[h], axis=0) for h in range(_H)]
    csum_flat = jnp.concatenate(csums)[None, :]
    out_ref[0] = jnp.broadcast_to(csum_flat, (_N, _F))
    diff = x_ref[0] - csum_flat
    loss_ref[...] += jnp.sum(diff * diff) * (1.0 / (_B * _N * _F))


def _hist_body(idx_hbm, counts_hbm, idx_v, zeros_v, ones_v, shared):
    cid = lax.axis_index("c")
    sid = lax.axis_index("s")
    slab = cid * _NS + sid
    h = slab % _H
    pltpu.sync_copy(idx_hbm.at[pl.ds(slab * _N, _N)], idx_v)

    def _zero(i, carry):
        zeros_v[pl.ds(i * _L, _L)] = jnp.zeros((_L,), jnp.float32)
        return carry

    lax.fori_loop(0, _N // _L, _zero, 0)
    ones_v[...] = jnp.full((_L,), 1.0, jnp.float32)
    pltpu.sync_copy(zeros_v, shared.at[pl.ds(sid * _N, _N)])
    plsc.subcore_barrier()

    base = jnp.full((_L,), h * _M, jnp.int32)

    def _accum(j, carry):
        iv = idx_v[pl.ds(j * _L, _L)] + base
        pltpu.sync_copy(ones_v, shared.at[iv], add=True)
        return carry

    lax.fori_loop(0, _N // _L, _accum, 0)
    plsc.subcore_barrier()

    @pl.when(sid < _H)
    def _():
        pltpu.sync_copy(shared.at[pl.ds(sid * _M, _M)],
                        counts_hbm.at[cid, sid])


def _finish_body(counts_ref, ema_ref, perp_ref, repl_ref):
    counts = counts_ref[0] + counts_ref[1]
    mean = counts * (1.0 / (_B * _N))
    ent = -jnp.sum(mean * jnp.log(mean + 1e-10), axis=1, keepdims=True)
    perp_ref[...] = jnp.broadcast_to(jnp.exp(ent), (_H, 128))
    expired = (ema_ref[...] < 2.0).astype(jnp.int32)
    repl_ref[...] = jnp.broadcast_to(
        jnp.sum(expired, axis=1, keepdims=True), (_H, 128))


def _run_main(x, codebooks, interpret=False):
    return pl.pallas_call(
        _main_body,
        grid=(_B,),
        in_specs=[
            pl.BlockSpec((1, _N, _F), lambda b: (b, 0, 0)),
            pl.BlockSpec((_H, _M, _D), lambda b: (0, 0, 0)),
        ],
        out_specs=pl.BlockSpec((1, _H, _N), lambda b: (b, 0, 0)),
        out_shape=jax.ShapeDtypeStruct((_B, _H, _N), jnp.int32),
        interpret=interpret,
    )(x, codebooks)


def _run_out_loss(x, codebooks, interpret=False):
    return pl.pallas_call(
        _out_loss_body,
        grid=(_B,),
        in_specs=[
            pl.BlockSpec((1, _N, _F), lambda b: (b, 0, 0)),
            pl.BlockSpec((_H, _M, _D), lambda b: (0, 0, 0)),
        ],
        out_specs=[
            pl.BlockSpec((1, _N, _F), lambda b: (b, 0, 0)),
            pl.BlockSpec((1, 128), lambda b: (0, 0)),
        ],
        out_shape=[
            jax.ShapeDtypeStruct((_B, _N, _F), jnp.float32),
            jax.ShapeDtypeStruct((1, 128), jnp.float32),
        ],
        interpret=interpret,
    )(x, codebooks)


def _run_hist(idx_flat):
    mesh = plsc.VectorSubcoreMesh(core_axis_name="c", subcore_axis_name="s")
    return pl.kernel(
        _hist_body,
        out_type=jax.ShapeDtypeStruct((_NC, _H, _M), jnp.float32),
        mesh=mesh,
        scratch_types=[
            pltpu.VMEM((_N,), jnp.int32),
            pltpu.VMEM((_N,), jnp.float32),
            pltpu.VMEM((_L,), jnp.float32),
            pltpu.VMEM_SHARED((_H * _M,), jnp.float32),
        ],
    )(idx_flat)


def _run_finish(counts2, ema, interpret=False):
    return pl.pallas_call(
        _finish_body,
        out_shape=[
            jax.ShapeDtypeStruct((_H, 128), jnp.float32),
            jax.ShapeDtypeStruct((_H, 128), jnp.int32),
        ],
        interpret=interpret,
    )(counts2, ema)


@jax.jit
def kernel(x, codebooks, ema_cluster_size):
    idx = _run_main(x, codebooks)
    counts2 = _run_hist(idx.reshape(_B * _H * _N))
    out, loss = _run_out_loss(x, codebooks)
    perp, repl = _run_finish(counts2, ema_cluster_size)
    return out, idx, loss[0, 0], perp[:, 0], repl[:, 0]


# speedup vs baseline: 1.1457x; 1.0891x over previous
import functools

import jax
import jax.numpy as jnp
from jax import lax
from jax.experimental import pallas as pl
from jax.experimental.pallas import tpu as pltpu
from jax.experimental.pallas import tpu_sc as plsc

_B, _N, _F = 4, 1024, 256
_H, _M, _D = 8, 2048, 32
_NC, _NS, _L = 2, 16, 16


def _main_body(x_ref, c_ref, idx_ref):
    iota = jax.lax.broadcasted_iota(jnp.int32, (_N, _M), 1)
    for h in range(_H):
        c = c_ref[h]
        l2_c = jnp.sum(c * c, axis=1)
        q = x_ref[0, :, h * _D:(h + 1) * _D]
        l2_q = jnp.sum(q * q, axis=1, keepdims=True)
        dot = jax.lax.dot_general(q, c, (((1,), (1,)), ((), ())),
                                  preferred_element_type=jnp.float32)
        sim = 2.0 * dot - (l2_q + l2_c[None, :])
        idx_ref[0, h, :] = jnp.argmax(sim, axis=-1).astype(jnp.int32)


def _out_loss_body(x_ref, c_ref, out_ref, loss_ref):
    b = pl.program_id(0)

    @pl.when(b == 0)
    def _():
        loss_ref[...] = jnp.zeros_like(loss_ref)

    csums = [jnp.sum(c_ref[h], axis=0) for h in range(_H)]
    csum_flat = jnp.concatenate(csums)[None, :]
    out_ref[0] = jnp.broadcast_to(csum_flat, (_N, _F))
    diff = x_ref[0] - csum_flat
    loss_ref[...] += jnp.sum(diff * diff) * (1.0 / (_B * _N * _F))


def _hist_body(idx_hbm, counts_hbm, idx_v, zeros_v, ones_v, shared):
    cid = lax.axis_index("c")
    sid = lax.axis_index("s")
    slab = cid * _NS + sid
    h = slab % _H
    pltpu.sync_copy(idx_hbm.at[pl.ds(slab * _N, _N)], idx_v)

    def _zero(i, carry):
        zeros_v[pl.ds(i * _L, _L)] = jnp.zeros((_L,), jnp.float32)
        return carry

    lax.fori_loop(0, _N // _L, _zero, 0)
    ones_v[...] = jnp.full((_L,), 1.0, jnp.float32)
    pltpu.sync_copy(zeros_v, shared.at[pl.ds(sid * _N, _N)])
    plsc.subcore_barrier()

    base = jnp.full((_L,), h * _M, jnp.int32)

    def _accum(j, carry):
        iv = idx_v[pl.ds(j * _L, _L)] + base
        pltpu.sync_copy(ones_v, shared.at[iv], add=True)
        return carry

    lax.fori_loop(0, _N // _L, _accum, 0)
    plsc.subcore_barrier()

    @pl.when(sid < _H)
    def _():
        pltpu.sync_copy(shared.at[pl.ds(sid * _M, _M)],
                        counts_hbm.at[cid, sid])


def _finish_body(counts_ref, ema_ref, perp_ref, repl_ref):
    counts = counts_ref[0] + counts_ref[1]
    mean = counts * (1.0 / (_B * _N))
    ent = -jnp.sum(mean * jnp.log(mean + 1e-10), axis=1, keepdims=True)
    perp_ref[...] = jnp.broadcast_to(jnp.exp(ent), (_H, 128))
    expired = (ema_ref[...] < 2.0).astype(jnp.int32)
    repl_ref[...] = jnp.broadcast_to(
        jnp.sum(expired, axis=1, keepdims=True), (_H, 128))


def _run_main(x, codebooks, interpret=False):
    return pl.pallas_call(
        _main_body,
        grid=(_B,),
        in_specs=[
            pl.BlockSpec((1, _N, _F), lambda b: (b, 0, 0)),
            pl.BlockSpec((_H, _M, _D), lambda b: (0, 0, 0)),
        ],
        out_specs=pl.BlockSpec((1, _H, _N), lambda b: (b, 0, 0)),
        out_shape=jax.ShapeDtypeStruct((_B, _H, _N), jnp.int32),
        interpret=interpret,
    )(x, codebooks)


def _run_out_loss(x, codebooks, interpret=False):
    return pl.pallas_call(
        _out_loss_body,
        grid=(_B,),
        in_specs=[
            pl.BlockSpec((1, _N, _F), lambda b: (b, 0, 0)),
            pl.BlockSpec((_H, _M, _D), lambda b: (0, 0, 0)),
        ],
        out_specs=[
            pl.BlockSpec((1, _N, _F), lambda b: (b, 0, 0)),
            pl.BlockSpec((1, 128), lambda b: (0, 0)),
        ],
        out_shape=[
            jax.ShapeDtypeStruct((_B, _N, _F), jnp.float32),
            jax.ShapeDtypeStruct((1, 128), jnp.float32),
        ],
        interpret=interpret,
    )(x, codebooks)


def _run_hist(idx_flat):
    mesh = plsc.VectorSubcoreMesh(core_axis_name="c", subcore_axis_name="s")
    return pl.kernel(
        _hist_body,
        out_type=jax.ShapeDtypeStruct((_NC, _H, _M), jnp.float32),
        mesh=mesh,
        scratch_types=[
            pltpu.VMEM((_N,), jnp.int32),
            pltpu.VMEM((_N,), jnp.float32),
            pltpu.VMEM((_L,), jnp.float32),
            pltpu.VMEM_SHARED((_H * _M,), jnp.float32),
        ],
    )(idx_flat)


def _run_finish(counts2, ema, interpret=False):
    return pl.pallas_call(
        _finish_body,
        out_shape=[
            jax.ShapeDtypeStruct((_H, 128), jnp.float32),
            jax.ShapeDtypeStruct((_H, 128), jnp.int32),
        ],
        interpret=interpret,
    )(counts2, ema)


@jax.jit
def kernel(x, codebooks, ema_cluster_size):
    idx = _run_main(x, codebooks)
    counts2 = _run_hist(idx.reshape(_B * _H * _N))
    out, loss = _run_out_loss(x, codebooks)
    perp, repl = _run_finish(counts2, ema_cluster_size)
    return out, idx, loss[0, 0], perp[:, 0], repl[:, 0]
